# Initial kernel scaffold; baseline (speedup 1.0000x reference)
import jax
import jax.numpy as jnp
from jax.experimental import pallas as pl


def kernel(x, router_scale, router_w, gating_w, linear_w, per_expert_scale):
    raise NotImplementedError("write your pallas kernel here")


# speedup vs baseline: 1.9143x; 1.9143x over previous
import functools

import jax
import jax.numpy as jnp
from jax import lax
from jax.experimental import pallas as pl

L, D, E, H = 2048, 768, 8, 1024
LT = 1024


def _router_weights(x, rs, rw, pes):
    var = jnp.mean(x * x, axis=-1, keepdims=True)
    ri = x * lax.rsqrt(var + 1e-6)
    ri = ri * lax.rsqrt(jnp.float32(D)) * rs
    logits = jnp.dot(ri, rw, preferred_element_type=jnp.float32)
    probs = jax.nn.softmax(logits, axis=-1)
    iota = lax.broadcasted_iota(jnp.int32, logits.shape, 1)
    m0 = jnp.max(logits, axis=-1, keepdims=True)
    i0 = jnp.min(jnp.where(logits == m0, iota, E), axis=-1, keepdims=True)
    l2 = jnp.where(iota == i0, -jnp.inf, logits)
    m1 = jnp.max(l2, axis=-1, keepdims=True)
    i1 = jnp.min(jnp.where(l2 == m1, iota, E), axis=-1, keepdims=True)
    top2 = (iota == i0) | (iota == i1)
    denom = jnp.sum(jnp.where(top2, probs, 0.0), axis=-1, keepdims=True)
    denom = jnp.where(denom > 0.0, denom, 1.0)
    return jnp.where(top2, probs / denom, 0.0) * pes, iota


def _moe_body(x_ref, rs_ref, rw_ref, gw_ref, lw_ref, pes_ref, out_ref):
    e = pl.program_id(1)
    x = x_ref[...]
    wdense, iota = _router_weights(x, rs_ref[...], rw_ref[...], pes_ref[...])
    wcol = jnp.sum(jnp.where(iota == e, wdense, 0.0), axis=-1, keepdims=True)

    g = gw_ref[0]
    g0 = lax.dot_general(x, g[0], (((1,), (1,)), ((), ())),
                         preferred_element_type=jnp.float32)
    g1 = lax.dot_general(x, g[1], (((1,), (1,)), ((), ())),
                         preferred_element_type=jnp.float32)
    act = jax.nn.gelu(g0) * g1
    y = lax.dot_general(act, lw_ref[0], (((1,), (0,)), ((), ())),
                        preferred_element_type=jnp.float32)
    contrib = y * wcol

    @pl.when(e == 0)
    def _():
        out_ref[...] = contrib

    @pl.when(e > 0)
    def _():
        out_ref[...] += contrib


@jax.jit
def kernel(x, router_scale, router_w, gating_w, linear_w, per_expert_scale):
    x2 = x.reshape(L, D)
    rs = router_scale.reshape(1, D)
    pes = per_expert_scale.reshape(1, E)
    out = pl.pallas_call(
        _moe_body,
        grid=(L // LT, E),
        in_specs=[
            pl.BlockSpec((LT, D), lambda lt, e: (lt, 0)),
            pl.BlockSpec((1, D), lambda lt, e: (0, 0)),
            pl.BlockSpec((D, E), lambda lt, e: (0, 0)),
            pl.BlockSpec((1, 2, H, D), lambda lt, e: (e, 0, 0, 0)),
            pl.BlockSpec((1, H, D), lambda lt, e: (e, 0, 0)),
            pl.BlockSpec((1, E), lambda lt, e: (0, 0)),
        ],
        out_specs=pl.BlockSpec((LT, D), lambda lt, e: (lt, 0)),
        out_shape=jax.ShapeDtypeStruct((L, D), jnp.float32),
    )(x2, rs, router_w, gating_w, linear_w, pes)
    return out.reshape(1, L, D)
